# baseline (speedup 1.0000x reference)
import functools

import jax
import jax.numpy as jnp
from jax import lax
from jax.experimental import pallas as pl
from jax.experimental.pallas import tpu as pltpu
from jax.experimental.pallas import tpu_sc as plsc

_N, _E, _D, _HID, _G = 10000, 320000, 128, 128, 64
_NP = 10240
_NC, _NS = 2, 16
_NW = _NC * _NS
_EPW = _E // _NW
_CHUNK = 125
_NCH = _EPW // _CHUNK
_RPT = _NP // _NS
_HCH = _NCH // 2


def _sc_body(x_hbm, srcr_hbm, dstr_hbm,
             part_hbm, cnt_hbm,
             acc_sh, cnt_sh, idx_s, idx_d, rows_a, rows_b,
             ones, sem_a, sem_b, sem_c):
    c = lax.axis_index("c")
    s = lax.axis_index("s")
    w = c * _NS + s

    def zfill(r, carry):
        for k in range(_D // 16):
            rows_a[r, pl.ds(k * 16, 16)] = jnp.zeros((16,), jnp.float32)
        return carry

    lax.fori_loop(0, 128, zfill, 0)
    for q in range(_RPT // 128):
        pltpu.sync_copy(rows_a,
                        acc_sh.at[pl.ds(s * _RPT + q * 128, 128)])
        pltpu.sync_copy(rows_a.at[0],
                        cnt_sh.at[pl.ds(s * _RPT + q * 128, 128)])

    for j in range(8):
        ones[pl.ds(j * 16, 16)] = jnp.ones((16,), jnp.float32)

    plsc.subcore_barrier()

    def scatter(j, rows):
        pltpu.sync_copy(rows, acc_sh.at[idx_d.at[j]], add=True)
        pltpu.async_copy(ones.at[pl.ds(0, _CHUNK)], cnt_sh.at[idx_d.at[j]],
                        sem_c, add=True)

    def chunk(t, carry):
        j0 = 2 * t
        ra = rows_a.at[pl.ds(0, _CHUNK)]
        rb = rows_b.at[pl.ds(0, _CHUNK)]
        pltpu.make_async_copy(x_hbm.at[idx_s.at[j0]], ra, sem_a).wait()
        pltpu.async_copy(x_hbm.at[idx_s.at[j0 + 1]], rb, sem_b)
        scatter(j0, ra)
        pltpu.make_async_copy(x_hbm.at[idx_s.at[j0 + 1]], rb, sem_b).wait()

        @pl.when(t + 1 < _HCH // 2)
        def _():
            pltpu.async_copy(x_hbm.at[idx_s.at[j0 + 2]], ra, sem_a)

        scatter(j0 + 1, rb)
        return carry

    def drain_cnt(j, carry):
        pltpu.make_async_copy(ones.at[pl.ds(0, _CHUNK)],
                              cnt_sh.at[idx_d.at[0]], sem_c).wait()
        return carry

    for half in range(2):
        pltpu.sync_copy(srcr_hbm.at[w, pl.ds(half * _HCH, _HCH)], idx_s)
        pltpu.sync_copy(dstr_hbm.at[w, pl.ds(half * _HCH, _HCH)], idx_d)
        pltpu.async_copy(x_hbm.at[idx_s.at[0]], rows_a.at[pl.ds(0, _CHUNK)], sem_a)
        lax.fori_loop(0, _HCH // 2, chunk, 0)
        lax.fori_loop(0, _HCH, drain_cnt, 0)

    plsc.subcore_barrier()

    pltpu.sync_copy(acc_sh.at[pl.ds(s * _RPT, _RPT)],
                    part_hbm.at[c, pl.ds(s * _RPT, _RPT)])

    @pl.when(s == 0)
    def _():
        pltpu.sync_copy(cnt_sh, cnt_hbm.at[c])


@functools.lru_cache(maxsize=None)
def _get_sc_aggregate():
    return pl.kernel(
        _sc_body,
        out_type=(
            jax.ShapeDtypeStruct((_NC, _NP, _D), jnp.float32),
            jax.ShapeDtypeStruct((_NC, _NP), jnp.float32),
        ),
        mesh=plsc.VectorSubcoreMesh(core_axis_name="c", subcore_axis_name="s"),
        scratch_types=[
            pltpu.VMEM_SHARED((_NP, _D), jnp.float32),
            pltpu.VMEM_SHARED((_NP,), jnp.float32),
            pltpu.VMEM((_HCH, _CHUNK), jnp.int32),
            pltpu.VMEM((_HCH, _CHUNK), jnp.int32),
            pltpu.VMEM((128, _D), jnp.float32),
            pltpu.VMEM((128, _D), jnp.float32),
            pltpu.VMEM((128,), jnp.float32),
            pltpu.SemaphoreType.DMA,
            pltpu.SemaphoreType.DMA,
            pltpu.SemaphoreType.DMA,
        ],
    )


_R = 512
_NB = _NP // _R


def _tc_body(part, cnt, x, bat, wl, bl, wr, w1, b1, w2, b2, out):
    i = pl.program_id(0)

    @pl.when(i == 0)
    def _():
        out[...] = jnp.full((_G, _D), -jnp.inf, jnp.float32)

    def matmul_t(a, b_ref):
        return lax.dot_general(a, b_ref[...], (((1,), (1,)), ((), ())),
                               preferred_element_type=jnp.float32)

    p = part[0] + part[1]
    c = jnp.maximum(cnt[0] + cnt[1], 1.0)
    mean = p / c[:, None]
    h = matmul_t(mean, wl) + bl[...] + matmul_t(x[...], wr)
    h = jnp.maximum(h, 0.0)
    h = matmul_t(h, w1) + b1[...]
    h = jnp.maximum(h, 0.0)
    h = matmul_t(h, w2) + b2[...]
    sig = jax.nn.sigmoid(h)

    bcol = bat[...]
    lo = bcol[0, 0]
    hi = jnp.minimum(bcol[_R - 1, 0], _G - 1)

    def pool(g, carry):
        vals = jnp.where(bcol == g, sig, -jnp.inf)
        red = jnp.max(vals, axis=0, keepdims=True)
        out[pl.ds(g, 1), :] = jnp.maximum(out[pl.ds(g, 1), :], red)
        return carry

    lax.fori_loop(lo, hi + 1, pool, 0)


def _tc_dense(part, cnt, x, bat3, wl_t, bl2, wr_t, w1_t, b12, w2_t, b22):
    return pl.pallas_call(
        _tc_body,
        grid=(_NB,),
        in_specs=[
            pl.BlockSpec((_NC, _R, _D), lambda i: (0, i, 0)),
            pl.BlockSpec((_NC, _R), lambda i: (0, i)),
            pl.BlockSpec((_R, _D), lambda i: (i, 0)),
            pl.BlockSpec((_R, 1), lambda i: (i, 0)),
            pl.BlockSpec((_D, _D), lambda i: (0, 0)),
            pl.BlockSpec((1, _D), lambda i: (0, 0)),
            pl.BlockSpec((_D, _D), lambda i: (0, 0)),
            pl.BlockSpec((512, _D), lambda i: (0, 0)),
            pl.BlockSpec((1, 512), lambda i: (0, 0)),
            pl.BlockSpec((_D, 512), lambda i: (0, 0)),
            pl.BlockSpec((1, _D), lambda i: (0, 0)),
        ],
        out_specs=pl.BlockSpec((_G, _D), lambda i: (0, 0)),
        out_shape=jax.ShapeDtypeStruct((_G, _D), jnp.float32),
    )(part, cnt, x, bat3, wl_t, bl2, wr_t, w1_t, b12, w2_t, b22)


def kernel(x, edge_index, batch, W_l, b_l, W_r, W1, b1, W2, b2):
    srcr = edge_index[0].reshape(_NW, _NCH, _CHUNK)
    dstr = edge_index[1].reshape(_NW, _NCH, _CHUNK)
    part, cnt = _get_sc_aggregate()(x, srcr, dstr)
    out = _tc_dense(
        part, cnt, x,
        jnp.pad(batch, (0, _NP - _N), constant_values=_G)[:, None],
        W_l, b_l.reshape(1, _D),
        W_r, W1, b1.reshape(1, 512), W2, b2.reshape(1, _D),
    )
    return out


# speedup vs baseline: 1.0681x; 1.0681x over previous
import functools

import jax
import jax.numpy as jnp
from jax import lax
from jax.experimental import pallas as pl
from jax.experimental.pallas import tpu as pltpu
from jax.experimental.pallas import tpu_sc as plsc

_N, _E, _D, _HID, _G = 10000, 320000, 128, 128, 64
_NP = 10240
_NC, _NS = 2, 16
_NW = _NC * _NS
_EPW = _E // _NW
_CHUNK = 125
_NCH = _EPW // _CHUNK
_RPT = _NP // _NS
_HCH = _NCH // 2


def _sc_body(x_hbm, eidx_hbm,
             part_hbm, cnt_hbm,
             acc_sh, cnt_sh, idx_s, idx_d, rows_a, rows_b,
             ones, sem_a, sem_b, sem_c, sem_sa, sem_sb):
    c = lax.axis_index("c")
    s = lax.axis_index("s")
    w = c * _NS + s

    def zfill(r, carry):
        for k in range(_D // 16):
            rows_a[r, pl.ds(k * 16, 16)] = jnp.zeros((16,), jnp.float32)
        return carry

    lax.fori_loop(0, 128, zfill, 0)
    for q in range(_RPT // 128):
        pltpu.sync_copy(rows_a,
                        acc_sh.at[pl.ds(s * _RPT + q * 128, 128)])
        pltpu.sync_copy(rows_a.at[0],
                        cnt_sh.at[pl.ds(s * _RPT + q * 128, 128)])

    for j in range(8):
        ones[pl.ds(j * 16, 16)] = jnp.ones((16,), jnp.float32)

    plsc.subcore_barrier()

    def scatter(j, rows, sem_s):
        pltpu.async_copy(rows, acc_sh.at[idx_d.at[j]], sem_s, add=True)
        pltpu.async_copy(ones.at[pl.ds(0, _CHUNK)], cnt_sh.at[idx_d.at[j]],
                        sem_c, add=True)

    def chunk(t, carry):
        j0 = 2 * t
        ra = rows_a.at[pl.ds(0, _CHUNK)]
        rb = rows_b.at[pl.ds(0, _CHUNK)]
        pltpu.make_async_copy(x_hbm.at[idx_s.at[j0]], ra, sem_a).wait()

        @pl.when(t > 0)
        def _():
            pltpu.make_async_copy(rb, acc_sh.at[idx_d.at[0]], sem_sb).wait()

        pltpu.async_copy(x_hbm.at[idx_s.at[j0 + 1]], rb, sem_b)
        scatter(j0, ra, sem_sa)
        pltpu.make_async_copy(x_hbm.at[idx_s.at[j0 + 1]], rb, sem_b).wait()
        pltpu.make_async_copy(ra, acc_sh.at[idx_d.at[0]], sem_sa).wait()

        @pl.when(t + 1 < _HCH // 2)
        def _():
            pltpu.async_copy(x_hbm.at[idx_s.at[j0 + 2]], ra, sem_a)

        scatter(j0 + 1, rb, sem_sb)
        return carry

    def drain_cnt(j, carry):
        pltpu.make_async_copy(ones.at[pl.ds(0, _CHUNK)],
                              cnt_sh.at[idx_d.at[0]], sem_c).wait()
        return carry

    for half in range(2):
        pltpu.sync_copy(eidx_hbm.at[0, w, pl.ds(half * _HCH, _HCH)], idx_s)
        pltpu.sync_copy(eidx_hbm.at[1, w, pl.ds(half * _HCH, _HCH)], idx_d)
        pltpu.async_copy(x_hbm.at[idx_s.at[0]], rows_a.at[pl.ds(0, _CHUNK)], sem_a)
        lax.fori_loop(0, _HCH // 2, chunk, 0)
        pltpu.make_async_copy(rows_b.at[pl.ds(0, _CHUNK)],
                              acc_sh.at[idx_d.at[0]], sem_sb).wait()
        lax.fori_loop(0, _HCH, drain_cnt, 0)

    plsc.subcore_barrier()

    pltpu.sync_copy(acc_sh.at[pl.ds(s * _RPT, _RPT)],
                    part_hbm.at[c, pl.ds(s * _RPT, _RPT)])

    @pl.when(s == 0)
    def _():
        pltpu.sync_copy(cnt_sh, cnt_hbm.at[c])


@functools.lru_cache(maxsize=None)
def _get_sc_aggregate():
    return pl.kernel(
        _sc_body,
        out_type=(
            jax.ShapeDtypeStruct((_NC, _NP, _D), jnp.float32),
            jax.ShapeDtypeStruct((_NC, _NP), jnp.float32),
        ),
        mesh=plsc.VectorSubcoreMesh(core_axis_name="c", subcore_axis_name="s"),
        scratch_types=[
            pltpu.VMEM_SHARED((_NP, _D), jnp.float32),
            pltpu.VMEM_SHARED((_NP,), jnp.float32),
            pltpu.VMEM((_HCH, _CHUNK), jnp.int32),
            pltpu.VMEM((_HCH, _CHUNK), jnp.int32),
            pltpu.VMEM((128, _D), jnp.float32),
            pltpu.VMEM((128, _D), jnp.float32),
            pltpu.VMEM((128,), jnp.float32),
            pltpu.SemaphoreType.DMA,
            pltpu.SemaphoreType.DMA,
            pltpu.SemaphoreType.DMA,
            pltpu.SemaphoreType.DMA,
            pltpu.SemaphoreType.DMA,
        ],
    )


_R = 1024
_NB = _NP // _R


def _tc_body(part, cnt, x, bat, wl, bl, wr, w1, b1, w2, b2, out):
    i = pl.program_id(0)

    @pl.when(i == 0)
    def _():
        out[...] = jnp.full((_G, _D), -jnp.inf, jnp.float32)

    def matmul_t(a, b_ref):
        return lax.dot_general(a, b_ref[...], (((1,), (1,)), ((), ())),
                               preferred_element_type=jnp.float32)

    p = part[0] + part[1]
    c = jnp.maximum(cnt[0] + cnt[1], 1.0)
    mean = p / c[:, None]
    h = matmul_t(mean, wl) + bl[...] + matmul_t(x[...], wr)
    h = jnp.maximum(h, 0.0)
    h = matmul_t(h, w1) + b1[...]
    h = jnp.maximum(h, 0.0)
    h = matmul_t(h, w2) + b2[...]
    sig = jax.nn.sigmoid(h)

    bcol = bat[...]
    lo = bcol[0, 0]
    hi = jnp.minimum(bcol[_R - 1, 0], _G - 1)

    def pool(g, carry):
        vals = jnp.where(bcol == g, sig, -jnp.inf)
        red = jnp.max(vals, axis=0, keepdims=True)
        out[pl.ds(g, 1), :] = jnp.maximum(out[pl.ds(g, 1), :], red)
        return carry

    lax.fori_loop(lo, hi + 1, pool, 0)


def _tc_dense(part, cnt, x, bat3, wl_t, bl2, wr_t, w1_t, b12, w2_t, b22):
    return pl.pallas_call(
        _tc_body,
        grid=(_NB,),
        in_specs=[
            pl.BlockSpec((_NC, _R, _D), lambda i: (0, i, 0)),
            pl.BlockSpec((_NC, _R), lambda i: (0, i)),
            pl.BlockSpec((_R, _D), lambda i: (i, 0)),
            pl.BlockSpec((_R, 1), lambda i: (i, 0)),
            pl.BlockSpec((_D, _D), lambda i: (0, 0)),
            pl.BlockSpec((1, _D), lambda i: (0, 0)),
            pl.BlockSpec((_D, _D), lambda i: (0, 0)),
            pl.BlockSpec((512, _D), lambda i: (0, 0)),
            pl.BlockSpec((1, 512), lambda i: (0, 0)),
            pl.BlockSpec((_D, 512), lambda i: (0, 0)),
            pl.BlockSpec((1, _D), lambda i: (0, 0)),
        ],
        out_specs=pl.BlockSpec((_G, _D), lambda i: (0, 0)),
        out_shape=jax.ShapeDtypeStruct((_G, _D), jnp.float32),
    )(part, cnt, x, bat3, wl_t, bl2, wr_t, w1_t, b12, w2_t, b22)


def kernel(x, edge_index, batch, W_l, b_l, W_r, W1, b1, W2, b2):
    eidx = edge_index.reshape(2, _NW, _NCH, _CHUNK)
    part, cnt = _get_sc_aggregate()(x, eidx)
    out = _tc_dense(
        part, cnt, x,
        jnp.pad(batch, (0, _NP - _N), constant_values=_G)[:, None],
        W_l, b_l.reshape(1, _D),
        W_r, W1, b1.reshape(1, 512), W2, b2.reshape(1, _D),
    )
    return out


# speedup vs baseline: 1.0699x; 1.0017x over previous
import functools

import jax
import jax.numpy as jnp
from jax import lax
from jax.experimental import pallas as pl
from jax.experimental.pallas import tpu as pltpu
from jax.experimental.pallas import tpu_sc as plsc

_N, _E, _D, _HID, _G = 10000, 320000, 128, 128, 64
_NP = 10240
_NC, _NS = 2, 16
_NW = _NC * _NS
_EPW = _E // _NW
_CHUNK = 125
_NCH = _EPW // _CHUNK
_RPT = _NP // _NS
_HCH = _NCH // 2


def _sc_body(x_hbm, eidx_hbm,
             part_hbm, cnt_hbm,
             acc_sh, cnt_sh, idx_s, idx_d, rows_a, rows_b,
             ones, sem_a, sem_b, sem_c):
    c = lax.axis_index("c")
    s = lax.axis_index("s")
    w = c * _NS + s

    def zfill(r, carry):
        for k in range(_D // 16):
            rows_a[r, pl.ds(k * 16, 16)] = jnp.zeros((16,), jnp.float32)
        return carry

    lax.fori_loop(0, 128, zfill, 0)
    for q in range(_RPT // 128):
        pltpu.sync_copy(rows_a,
                        acc_sh.at[pl.ds(s * _RPT + q * 128, 128)])
        pltpu.sync_copy(rows_a.at[0],
                        cnt_sh.at[pl.ds(s * _RPT + q * 128, 128)])

    for j in range(8):
        ones[pl.ds(j * 16, 16)] = jnp.ones((16,), jnp.float32)

    plsc.subcore_barrier()

    def scatter(j, rows):
        pltpu.sync_copy(rows, acc_sh.at[idx_d.at[j]], add=True)
        pltpu.async_copy(ones.at[pl.ds(0, _CHUNK)], cnt_sh.at[idx_d.at[j]],
                        sem_c, add=True)

    def chunk(t, carry):
        j0 = 2 * t
        ra = rows_a.at[pl.ds(0, _CHUNK)]
        rb = rows_b.at[pl.ds(0, _CHUNK)]
        pltpu.make_async_copy(x_hbm.at[idx_s.at[j0]], ra, sem_a).wait()
        pltpu.async_copy(x_hbm.at[idx_s.at[j0 + 1]], rb, sem_b)
        scatter(j0, ra)
        pltpu.make_async_copy(x_hbm.at[idx_s.at[j0 + 1]], rb, sem_b).wait()

        @pl.when(t + 1 < _HCH // 2)
        def _():
            pltpu.async_copy(x_hbm.at[idx_s.at[j0 + 2]], ra, sem_a)

        scatter(j0 + 1, rb)
        return carry

    def drain_cnt(j, carry):
        pltpu.make_async_copy(ones.at[pl.ds(0, _CHUNK)],
                              cnt_sh.at[idx_d.at[0]], sem_c).wait()
        return carry

    for half in range(2):
        pltpu.sync_copy(eidx_hbm.at[0, w, pl.ds(half * _HCH, _HCH)], idx_s)
        pltpu.sync_copy(eidx_hbm.at[1, w, pl.ds(half * _HCH, _HCH)], idx_d)
        pltpu.async_copy(x_hbm.at[idx_s.at[0]], rows_a.at[pl.ds(0, _CHUNK)], sem_a)
        lax.fori_loop(0, _HCH // 2, chunk, 0)
        lax.fori_loop(0, _HCH, drain_cnt, 0)

    plsc.subcore_barrier()

    pltpu.sync_copy(acc_sh.at[pl.ds(s * _RPT, _RPT)],
                    part_hbm.at[c, pl.ds(s * _RPT, _RPT)])

    @pl.when(s == 0)
    def _():
        pltpu.sync_copy(cnt_sh, cnt_hbm.at[c])


@functools.lru_cache(maxsize=None)
def _get_sc_aggregate():
    return pl.kernel(
        _sc_body,
        out_type=(
            jax.ShapeDtypeStruct((_NC, _NP, _D), jnp.float32),
            jax.ShapeDtypeStruct((_NC, _NP), jnp.float32),
        ),
        mesh=plsc.VectorSubcoreMesh(core_axis_name="c", subcore_axis_name="s"),
        scratch_types=[
            pltpu.VMEM_SHARED((_NP, _D), jnp.float32),
            pltpu.VMEM_SHARED((_NP,), jnp.float32),
            pltpu.VMEM((_HCH, _CHUNK), jnp.int32),
            pltpu.VMEM((_HCH, _CHUNK), jnp.int32),
            pltpu.VMEM((128, _D), jnp.float32),
            pltpu.VMEM((128, _D), jnp.float32),
            pltpu.VMEM((128,), jnp.float32),
            pltpu.SemaphoreType.DMA,
            pltpu.SemaphoreType.DMA,
            pltpu.SemaphoreType.DMA,
        ],
    )


_R = 1024
_NB = _NP // _R


def _tc_body(part, cnt, x, bat, wl, bl, wr, w1, b1, w2, b2, out):
    i = pl.program_id(0)

    @pl.when(i == 0)
    def _():
        out[...] = jnp.full((_G, _D), -jnp.inf, jnp.float32)

    def matmul_t(a, b_ref):
        return lax.dot_general(a.astype(jnp.bfloat16),
                               b_ref[...].astype(jnp.bfloat16),
                               (((1,), (1,)), ((), ())),
                               preferred_element_type=jnp.float32)

    p = part[0] + part[1]
    c = jnp.maximum(cnt[0] + cnt[1], 1.0)
    mean = p / c[:, None]
    h = matmul_t(mean, wl) + bl[...] + matmul_t(x[...], wr)
    h = jnp.maximum(h, 0.0)
    h = matmul_t(h, w1) + b1[...]
    h = jnp.maximum(h, 0.0)
    h = matmul_t(h, w2) + b2[...]
    sig = jax.nn.sigmoid(h)

    bcol = bat[...]
    lo = bcol[0, 0]
    hi = jnp.minimum(bcol[_R - 1, 0], _G - 1)

    def pool(g, carry):
        vals = jnp.where(bcol == g, sig, -jnp.inf)
        red = jnp.max(vals, axis=0, keepdims=True)
        out[pl.ds(g, 1), :] = jnp.maximum(out[pl.ds(g, 1), :], red)
        return carry

    lax.fori_loop(lo, hi + 1, pool, 0)


def _tc_dense(part, cnt, x, bat3, wl_t, bl2, wr_t, w1_t, b12, w2_t, b22):
    return pl.pallas_call(
        _tc_body,
        grid=(_NB,),
        in_specs=[
            pl.BlockSpec((_NC, _R, _D), lambda i: (0, i, 0)),
            pl.BlockSpec((_NC, _R), lambda i: (0, i)),
            pl.BlockSpec((_R, _D), lambda i: (i, 0)),
            pl.BlockSpec((_R, 1), lambda i: (i, 0)),
            pl.BlockSpec((_D, _D), lambda i: (0, 0)),
            pl.BlockSpec((1, _D), lambda i: (0, 0)),
            pl.BlockSpec((_D, _D), lambda i: (0, 0)),
            pl.BlockSpec((512, _D), lambda i: (0, 0)),
            pl.BlockSpec((1, 512), lambda i: (0, 0)),
            pl.BlockSpec((_D, 512), lambda i: (0, 0)),
            pl.BlockSpec((1, _D), lambda i: (0, 0)),
        ],
        out_specs=pl.BlockSpec((_G, _D), lambda i: (0, 0)),
        out_shape=jax.ShapeDtypeStruct((_G, _D), jnp.float32),
    )(part, cnt, x, bat3, wl_t, bl2, wr_t, w1_t, b12, w2_t, b22)


def kernel(x, edge_index, batch, W_l, b_l, W_r, W1, b1, W2, b2):
    eidx = edge_index.reshape(2, _NW, _NCH, _CHUNK)
    part, cnt = _get_sc_aggregate()(x, eidx)
    out = _tc_dense(
        part, cnt, x,
        jnp.pad(batch, (0, _NP - _N), constant_values=_G)[:, None],
        W_l, b_l.reshape(1, _D),
        W_r, W1, b1.reshape(1, 512), W2, b2.reshape(1, _D),
    )
    return out
